# Initial kernel scaffold; baseline (speedup 1.0000x reference)
import jax
import jax.numpy as jnp
from jax.experimental import pallas as pl


def kernel(x, expert_indices, expert_weights, W_up, W_down):
    raise NotImplementedError("write your pallas kernel here")


# speedup vs baseline: 2.0114x; 2.0114x over previous
import jax
import jax.numpy as jnp
from jax.experimental import pallas as pl
from jax.experimental.pallas import tpu as pltpu

_BETA = 0.9
_VTH = 1.0
_MB = 128

_INTERPRET = False


def _ffn_body(be_ref, xg_ref, wup_ref, wdn_ref, w_ref, out_ref):
    wup = wup_ref[0]
    wdn = wdn_ref[0]
    wrow = w_ref[:, 0:1]
    tsteps, mb, _ = xg_ref.shape
    f = wup.shape[0]
    dm = wdn.shape[0]
    v1 = jnp.zeros((mb, f), jnp.float32)
    v2 = jnp.zeros((mb, dm), jnp.float32)
    for t in range(tsteps):
        h = jax.lax.dot_general(
            xg_ref[t], wup, (((1,), (1,)), ((), ())),
            preferred_element_type=jnp.float32)
        v1 = _BETA * v1 + h
        s1 = (v1 >= _VTH).astype(jnp.float32)
        v1 = v1 - s1 * _VTH
        o = jax.lax.dot_general(
            s1, wdn, (((1,), (1,)), ((), ())),
            preferred_element_type=jnp.float32)
        v2 = _BETA * v2 + o
        s2 = (v2 >= _VTH).astype(jnp.float32)
        v2 = v2 - s2 * _VTH
        out_ref[t] = s2 * wrow


def _routing(expert_indices, expert_weights, n_experts, mb):
    n, k = expert_indices.shape
    p = n * k
    flat_e = expert_indices.reshape(-1).astype(jnp.int32)
    w_flat = expert_weights.reshape(-1)
    order = jnp.argsort(flat_e, stable=True).astype(jnp.int32)
    sorted_e = flat_e[order]
    counts = jnp.zeros((n_experts,), jnp.int32).at[flat_e].add(1)
    csum_excl = jnp.cumsum(counts) - counts
    padded = ((counts + mb - 1) // mb) * mb
    ends = jnp.cumsum(padded)
    starts = ends - padded
    r_pad = p + n_experts * mb
    g = r_pad // mb
    padded_pos = starts[sorted_e] + jnp.arange(p, dtype=jnp.int32) - csum_excl[sorted_e]
    row_src = jnp.zeros((r_pad,), jnp.int32).at[padded_pos].set(order // k)
    row_w = jnp.zeros((r_pad,), jnp.float32).at[padded_pos].set(w_flat[order])
    q = jnp.zeros((p,), jnp.int32).at[order].set(padded_pos).reshape(n, k)
    gstart = jnp.arange(g, dtype=jnp.int32) * mb
    block_expert = jnp.minimum(
        jnp.sum((gstart[:, None] >= ends[None, :]).astype(jnp.int32), axis=1),
        n_experts - 1).astype(jnp.int32)
    return row_src, row_w, q, block_expert, r_pad, g


def kernel(x, expert_indices, expert_weights, W_up, W_down):
    tsteps, n, d = x.shape
    e, f, _ = W_up.shape
    k = expert_indices.shape[1]
    mb = _MB

    row_src, row_w, q, block_expert, r_pad, g = _routing(
        expert_indices, expert_weights, e, mb)

    xg = jnp.take(x, row_src, axis=1)

    w_exp = jnp.broadcast_to(row_w[:, None], (r_pad, 128))

    grid_spec = pltpu.PrefetchScalarGridSpec(
        num_scalar_prefetch=1,
        grid=(g,),
        in_specs=[
            pl.BlockSpec((tsteps, mb, d), lambda i, be: (0, i, 0)),
            pl.BlockSpec((1, f, d), lambda i, be: (be[i], 0, 0)),
            pl.BlockSpec((1, d, f), lambda i, be: (be[i], 0, 0)),
            pl.BlockSpec((mb, 128), lambda i, be: (i, 0)),
        ],
        out_specs=pl.BlockSpec((tsteps, mb, d), lambda i, be: (0, i, 0)),
    )
    og = pl.pallas_call(
        _ffn_body,
        grid_spec=grid_spec,
        out_shape=jax.ShapeDtypeStruct((tsteps, r_pad, d), jnp.float32),
        interpret=_INTERPRET,
    )(block_expert, xg, W_up, W_down, w_exp)

    out = jnp.take(og, q[:, 0], axis=1)
    for j in range(1, k):
        out = out + jnp.take(og, q[:, j], axis=1)
    return out


# speedup vs baseline: 2.0184x; 1.0035x over previous
import jax
import jax.numpy as jnp
from jax.experimental import pallas as pl
from jax.experimental.pallas import tpu as pltpu

_BETA = 0.9
_VTH = 1.0
_MB = 128

_INTERPRET = False


def _ffn_body(be_ref, bv_ref, xg_ref, wup_ref, wdn_ref, w_ref, out_ref):
    i = pl.program_id(0)

    @pl.when(bv_ref[i] == 1)
    def _():
        wup = wup_ref[0]
        wdn = wdn_ref[0]
        wrow = w_ref[:, 0:1]
        tsteps, mb, _ = xg_ref.shape
        f = wup.shape[0]
        dm = wdn.shape[0]
        v1 = jnp.zeros((mb, f), jnp.float32)
        v2 = jnp.zeros((mb, dm), jnp.float32)
        for t in range(tsteps):
            h = jax.lax.dot_general(
                xg_ref[t], wup, (((1,), (1,)), ((), ())),
                preferred_element_type=jnp.float32)
            v1 = _BETA * v1 + h
            s1 = (v1 >= _VTH).astype(jnp.float32)
            v1 = v1 - s1 * _VTH
            o = jax.lax.dot_general(
                s1, wdn, (((1,), (1,)), ((), ())),
                preferred_element_type=jnp.float32)
            v2 = _BETA * v2 + o
            s2 = (v2 >= _VTH).astype(jnp.float32)
            v2 = v2 - s2 * _VTH
            out_ref[t] = s2 * wrow


def _routing(expert_indices, expert_weights, n_experts, mb):
    n, k = expert_indices.shape
    p = n * k
    flat_e = expert_indices.reshape(-1).astype(jnp.int32)
    w_flat = expert_weights.reshape(-1)
    order = jnp.argsort(flat_e, stable=True).astype(jnp.int32)
    sorted_e = flat_e[order]
    counts = jnp.zeros((n_experts,), jnp.int32).at[flat_e].add(1)
    csum_excl = jnp.cumsum(counts) - counts
    padded = ((counts + mb - 1) // mb) * mb
    ends = jnp.cumsum(padded)
    starts = ends - padded
    r_pad = p + n_experts * mb
    g = r_pad // mb
    padded_pos = starts[sorted_e] + jnp.arange(p, dtype=jnp.int32) - csum_excl[sorted_e]
    row_src = jnp.zeros((r_pad,), jnp.int32).at[padded_pos].set(order // k)
    row_w = jnp.zeros((r_pad,), jnp.float32).at[padded_pos].set(w_flat[order])
    q = jnp.zeros((p,), jnp.int32).at[order].set(padded_pos).reshape(n, k)
    gstart = jnp.arange(g, dtype=jnp.int32) * mb
    block_expert = jnp.minimum(
        jnp.sum((gstart[:, None] >= ends[None, :]).astype(jnp.int32), axis=1),
        n_experts - 1).astype(jnp.int32)
    block_valid = (gstart < ends[-1]).astype(jnp.int32)
    return row_src, row_w, q, block_expert, block_valid, r_pad, g


def kernel(x, expert_indices, expert_weights, W_up, W_down):
    tsteps, n, d = x.shape
    e, f, _ = W_up.shape
    k = expert_indices.shape[1]
    mb = _MB

    row_src, row_w, q, block_expert, block_valid, r_pad, g = _routing(
        expert_indices, expert_weights, e, mb)

    xg = jnp.take(x, row_src, axis=1)

    w_exp = jnp.broadcast_to(row_w[:, None], (r_pad, 128))

    grid_spec = pltpu.PrefetchScalarGridSpec(
        num_scalar_prefetch=2,
        grid=(g,),
        in_specs=[
            pl.BlockSpec((tsteps, mb, d), lambda i, be, bv: (0, i, 0)),
            pl.BlockSpec((1, f, d), lambda i, be, bv: (be[i], 0, 0)),
            pl.BlockSpec((1, d, f), lambda i, be, bv: (be[i], 0, 0)),
            pl.BlockSpec((mb, 128), lambda i, be, bv: (i, 0)),
        ],
        out_specs=pl.BlockSpec((tsteps, mb, d), lambda i, be, bv: (0, i, 0)),
    )
    og = pl.pallas_call(
        _ffn_body,
        grid_spec=grid_spec,
        out_shape=jax.ShapeDtypeStruct((tsteps, r_pad, d), jnp.float32),
        interpret=_INTERPRET,
    )(block_expert, block_valid, xg, W_up, W_down, w_exp)

    out = jnp.take(og, q[:, 0], axis=1)
    for j in range(1, k):
        out = out + jnp.take(og, q[:, j], axis=1)
    return out


# speedup vs baseline: 3.6486x; 1.8077x over previous
import functools

import jax
import jax.numpy as jnp
from jax import lax
from jax.experimental import pallas as pl
from jax.experimental.pallas import tpu as pltpu
from jax.experimental.pallas import tpu_sc as plsc

_BETA = 0.9
_VTH = 1.0
_MB = 128


def _ffn_body(be_ref, bv_ref, xg_ref, wup_ref, wdn_ref, w_ref, out_ref):
    i = pl.program_id(0)

    @pl.when(bv_ref[i] == 1)
    def _():
        wup = wup_ref[0]
        wdn = wdn_ref[0]
        wrow = w_ref[:, 0:1]
        tsteps, mb, _ = xg_ref.shape
        f = wup.shape[0]
        dm = wdn.shape[0]
        v1 = jnp.zeros((mb, f), jnp.float32)
        v2 = jnp.zeros((mb, dm), jnp.float32)
        for t in range(tsteps):
            h = jax.lax.dot_general(
                xg_ref[t], wup, (((1,), (1,)), ((), ())),
                preferred_element_type=jnp.float32)
            v1 = _BETA * v1 + h
            s1 = (v1 >= _VTH).astype(jnp.float32)
            v1 = v1 - s1 * _VTH
            o = jax.lax.dot_general(
                s1, wdn, (((1,), (1,)), ((), ())),
                preferred_element_type=jnp.float32)
            v2 = _BETA * v2 + o
            s2 = (v2 >= _VTH).astype(jnp.float32)
            v2 = v2 - s2 * _VTH
            out_ref[t] = s2 * wrow


def _routing(expert_indices, expert_weights, n_experts, mb):
    n, k = expert_indices.shape
    p = n * k
    flat_e = expert_indices.reshape(-1).astype(jnp.int32)
    w_flat = expert_weights.reshape(-1)
    onehot = (flat_e[:, None] == jnp.arange(n_experts, dtype=jnp.int32)[None, :]
              ).astype(jnp.int32)
    csum = jnp.cumsum(onehot, axis=0)
    counts = csum[-1]
    rank = jnp.take_along_axis(csum, flat_e[:, None], axis=1)[:, 0] - 1
    padded = ((counts + mb - 1) // mb) * mb
    ends = jnp.cumsum(padded)
    starts = ends - padded
    r_pad = p + n_experts * mb
    g = r_pad // mb
    q_flat = starts[flat_e] + rank
    row_src = jnp.zeros((r_pad,), jnp.int32).at[q_flat].set(
        jnp.arange(p, dtype=jnp.int32) // k)
    row_w = jnp.zeros((r_pad,), jnp.float32).at[q_flat].set(w_flat)
    q = q_flat.reshape(n, k)
    gstart = jnp.arange(g, dtype=jnp.int32) * mb
    block_expert = jnp.minimum(
        jnp.sum((gstart[:, None] >= ends[None, :]).astype(jnp.int32), axis=1),
        n_experts - 1).astype(jnp.int32)
    block_valid = (gstart < ends[-1]).astype(jnp.int32)
    return row_src, row_w, q, block_expert, block_valid, r_pad, g


def _sc_gather(table, idx, chunk):
    nrows = idx.shape[0]
    d = table.shape[1]
    info = plsc.get_sparse_core_info()
    nc = info.num_cores
    nw = nc * info.num_subcores
    per_w = nrows // nw
    assert per_w * nw == nrows and per_w % chunk == 0
    nch = per_w // chunk
    mesh = plsc.VectorSubcoreMesh(core_axis_name="c", subcore_axis_name="s")

    @functools.partial(
        pl.kernel, mesh=mesh,
        out_type=jax.ShapeDtypeStruct((nrows, d), jnp.float32),
        scratch_types=[
            pltpu.VMEM((chunk,), jnp.int32),
            pltpu.VMEM((chunk, d), jnp.float32),
            pltpu.SemaphoreType.DMA,
        ])
    def k(table_hbm, idx_hbm, out_hbm, idx_v, rows_v, sem):
        wid = lax.axis_index("s") * nc + lax.axis_index("c")
        base = wid * per_w

        def body(c, carry):
            off = base + c * chunk
            pltpu.sync_copy(idx_hbm.at[pl.ds(off, chunk)], idx_v)
            pltpu.async_copy(table_hbm.at[idx_v], rows_v, sem).wait()
            pltpu.sync_copy(rows_v, out_hbm.at[pl.ds(off, chunk)])
            return carry

        lax.fori_loop(0, nch, body, 0)

    return k(table, idx)


def _sc_combine(table, idxs, chunk):
    kk, nrows = idxs.shape
    d = table.shape[1]
    info = plsc.get_sparse_core_info()
    nc = info.num_cores
    nw = nc * info.num_subcores
    per_w = nrows // nw
    assert per_w * nw == nrows and per_w % chunk == 0
    nch = per_w // chunk
    mesh = plsc.VectorSubcoreMesh(core_axis_name="c", subcore_axis_name="s")

    @functools.partial(
        pl.kernel, mesh=mesh,
        out_type=jax.ShapeDtypeStruct((nrows, d), jnp.float32),
        scratch_types=[
            pltpu.VMEM((chunk,), jnp.int32),
            pltpu.VMEM((chunk, d), jnp.float32),
            pltpu.VMEM((chunk, d), jnp.float32),
            pltpu.SemaphoreType.DMA,
        ])
    def k(table_hbm, idx_hbm, out_hbm, idx_v, acc_v, row_v, sem):
        wid = lax.axis_index("s") * nc + lax.axis_index("c")
        base = wid * per_w

        def body(c, carry):
            off = base + c * chunk
            pltpu.sync_copy(idx_hbm.at[0, pl.ds(off, chunk)], idx_v)
            pltpu.async_copy(table_hbm.at[idx_v], acc_v, sem).wait()
            for j in range(1, kk):
                pltpu.sync_copy(idx_hbm.at[j, pl.ds(off, chunk)], idx_v)
                pltpu.async_copy(table_hbm.at[idx_v], row_v, sem).wait()

                def radd(i, cr):
                    for col in range(d // 16):
                        sl = pl.ds(col * 16, 16)
                        acc_v[i, sl] = acc_v[i, sl] + row_v[i, sl]
                    return cr

                lax.fori_loop(0, chunk, radd, 0)
            pltpu.sync_copy(acc_v, out_hbm.at[pl.ds(off, chunk)])
            return carry

        lax.fori_loop(0, nch, body, 0)

    return k(table, idxs)


def kernel(x, expert_indices, expert_weights, W_up, W_down):
    tsteps, n, d = x.shape
    e, f, _ = W_up.shape
    k = expert_indices.shape[1]
    mb = _MB

    row_src, row_w, q, block_expert, block_valid, r_pad, g = _routing(
        expert_indices, expert_weights, e, mb)

    x2 = x.reshape(tsteps * n, d)
    xg_idx = ((jnp.arange(tsteps, dtype=jnp.int32) * n)[:, None]
              + row_src[None, :]).reshape(-1)
    xg = _sc_gather(x2, xg_idx, 64).reshape(tsteps, r_pad, d)

    w_exp = jnp.broadcast_to(row_w[:, None], (r_pad, 128))

    grid_spec = pltpu.PrefetchScalarGridSpec(
        num_scalar_prefetch=2,
        grid=(g,),
        in_specs=[
            pl.BlockSpec((tsteps, mb, d), lambda i, be, bv: (0, i, 0)),
            pl.BlockSpec((1, f, d), lambda i, be, bv: (be[i], 0, 0)),
            pl.BlockSpec((1, d, f), lambda i, be, bv: (be[i], 0, 0)),
            pl.BlockSpec((mb, 128), lambda i, be, bv: (i, 0)),
        ],
        out_specs=pl.BlockSpec((tsteps, mb, d), lambda i, be, bv: (0, i, 0)),
    )
    og = pl.pallas_call(
        _ffn_body,
        grid_spec=grid_spec,
        out_shape=jax.ShapeDtypeStruct((tsteps, r_pad, d), jnp.float32),
    )(block_expert, block_valid, xg, W_up, W_down, w_exp)

    og2 = og.reshape(tsteps * r_pad, d)
    c_idx = jnp.stack([
        ((jnp.arange(tsteps, dtype=jnp.int32) * r_pad)[:, None]
         + q[None, :, j]).reshape(-1)
        for j in range(k)])
    out2 = _sc_combine(og2, c_idx, 32)
    return out2.reshape(tsteps, n, d)


# speedup vs baseline: 3.8456x; 1.0540x over previous
import functools

import jax
import jax.numpy as jnp
from jax import lax
from jax.experimental import pallas as pl
from jax.experimental.pallas import tpu as pltpu
from jax.experimental.pallas import tpu_sc as plsc

_BETA = 0.9
_VTH = 1.0
_MB = 128


def _ffn_body(be_ref, bv_ref, xg_ref, wup_ref, wdn_ref, w_ref, out_ref):
    i = pl.program_id(0)

    @pl.when(bv_ref[i] == 1)
    def _():
        wup = wup_ref[0]
        wdn = wdn_ref[0]
        wrow = w_ref[:, 0:1]
        tsteps, mb, _ = xg_ref.shape
        f = wup.shape[0]
        dm = wdn.shape[0]
        v1 = jnp.zeros((mb, f), jnp.float32)
        v2 = jnp.zeros((mb, dm), jnp.float32)
        for t in range(tsteps):
            h = jax.lax.dot_general(
                xg_ref[t], wup, (((1,), (1,)), ((), ())),
                preferred_element_type=jnp.float32)
            v1 = _BETA * v1 + h
            s1 = (v1 >= _VTH).astype(jnp.float32)
            v1 = v1 - s1 * _VTH
            o = jax.lax.dot_general(
                s1, wdn, (((1,), (1,)), ((), ())),
                preferred_element_type=jnp.float32)
            v2 = _BETA * v2 + o
            s2 = (v2 >= _VTH).astype(jnp.float32)
            v2 = v2 - s2 * _VTH
            out_ref[t] = s2 * wrow


def _routing(expert_indices, expert_weights, n_experts, mb):
    n, k = expert_indices.shape
    p = n * k
    flat_e = expert_indices.reshape(-1).astype(jnp.int32)
    w_flat = expert_weights.reshape(-1)
    onehot = (flat_e[:, None] == jnp.arange(n_experts, dtype=jnp.int32)[None, :]
              ).astype(jnp.int32)
    csum = jnp.cumsum(onehot, axis=0)
    counts = csum[-1]
    rank = jnp.take_along_axis(csum, flat_e[:, None], axis=1)[:, 0] - 1
    padded = ((counts + mb - 1) // mb) * mb
    ends = jnp.cumsum(padded)
    starts = ends - padded
    r_pad = p + n_experts * mb
    g = r_pad // mb
    q_flat = starts[flat_e] + rank
    row_src = jnp.zeros((r_pad,), jnp.int32).at[q_flat].set(
        jnp.arange(p, dtype=jnp.int32) // k)
    row_w = jnp.zeros((r_pad,), jnp.float32).at[q_flat].set(w_flat)
    q = q_flat.reshape(n, k)
    gstart = jnp.arange(g, dtype=jnp.int32) * mb
    block_expert = jnp.minimum(
        jnp.sum((gstart[:, None] >= ends[None, :]).astype(jnp.int32), axis=1),
        n_experts - 1).astype(jnp.int32)
    block_valid = (gstart < ends[-1]).astype(jnp.int32)
    return row_src, row_w, q, block_expert, block_valid, r_pad, g


def _sc_gather(table, idx, chunk):
    nrows = idx.shape[0]
    d = table.shape[1]
    info = plsc.get_sparse_core_info()
    nc = info.num_cores
    nw = nc * info.num_subcores
    per_w = nrows // nw
    assert per_w * nw == nrows and per_w % (2 * chunk) == 0
    nch = per_w // chunk
    mesh = plsc.VectorSubcoreMesh(core_axis_name="c", subcore_axis_name="s")

    @functools.partial(
        pl.kernel, mesh=mesh,
        out_type=jax.ShapeDtypeStruct((nrows, d), jnp.float32),
        scratch_types=[
            pltpu.VMEM((chunk,), jnp.int32),
            pltpu.VMEM((chunk,), jnp.int32),
            pltpu.VMEM((chunk, d), jnp.float32),
            pltpu.VMEM((chunk, d), jnp.float32),
            pltpu.SemaphoreType.DMA,
            pltpu.SemaphoreType.DMA,
        ])
    def k(table_hbm, idx_hbm, out_hbm, idx_a, idx_b, rows_a, rows_b,
          sem_a, sem_b):
        wid = lax.axis_index("s") * nc + lax.axis_index("c")
        base = wid * per_w
        bufs = ((idx_a, rows_a, sem_a), (idx_b, rows_b, sem_b))

        def issue(c, b):
            iv, rv, sem = bufs[b]
            pltpu.sync_copy(idx_hbm.at[pl.ds(base + c * chunk, chunk)], iv)
            pltpu.async_copy(table_hbm.at[iv], rv, sem)

        issue(0, 0)

        def body(go, carry):
            for b in range(2):
                c = go * 2 + b
                iv, rv, sem = bufs[b]
                pltpu.make_async_copy(table_hbm.at[iv], rv, sem).wait()

                @pl.when(c + 1 < nch)
                def _():
                    nv, rv2, sem2 = bufs[1 - b]
                    pltpu.sync_copy(
                        idx_hbm.at[pl.ds(base + (c + 1) * chunk, chunk)], nv)
                    pltpu.async_copy(table_hbm.at[nv], rv2, sem2)

                pltpu.sync_copy(rv, out_hbm.at[pl.ds(base + c * chunk, chunk)])
            return carry

        lax.fori_loop(0, nch // 2, body, 0)

    return k(table, idx)


def _sc_combine(table, idxs, chunk):
    kk, nrows = idxs.shape
    d = table.shape[1]
    info = plsc.get_sparse_core_info()
    nc = info.num_cores
    nw = nc * info.num_subcores
    per_w = nrows // nw
    assert per_w * nw == nrows and per_w % (2 * chunk) == 0
    nch = per_w // chunk
    mesh = plsc.VectorSubcoreMesh(core_axis_name="c", subcore_axis_name="s")

    @functools.partial(
        pl.kernel, mesh=mesh,
        out_type=jax.ShapeDtypeStruct((nrows, d), jnp.float32),
        scratch_types=(
            [pltpu.VMEM((chunk,), jnp.int32) for _ in range(2 * kk)]
            + [pltpu.VMEM((chunk, d), jnp.float32) for _ in range(2 * kk)]
            + [pltpu.SemaphoreType.DMA for _ in range(2 * kk)]
        ))
    def k(table_hbm, idx_hbm, out_hbm, *scratch):
        ivs = scratch[:2 * kk]
        rvs = scratch[2 * kk:4 * kk]
        sems = scratch[4 * kk:]
        wid = lax.axis_index("s") * nc + lax.axis_index("c")
        base = wid * per_w

        def issue(c, b):
            for j in range(kk):
                iv = ivs[b * kk + j]
                rv = rvs[b * kk + j]
                sem = sems[b * kk + j]
                pltpu.sync_copy(idx_hbm.at[j, pl.ds(base + c * chunk, chunk)], iv)
                pltpu.async_copy(table_hbm.at[iv], rv, sem)

        issue(0, 0)

        def body(go, carry):
            for b in range(2):
                c = go * 2 + b
                for j in range(kk):
                    pltpu.make_async_copy(
                        table_hbm.at[ivs[b * kk + j]], rvs[b * kk + j],
                        sems[b * kk + j]).wait()

                @pl.when(c + 1 < nch)
                def _():
                    issue(c + 1, 1 - b)

                acc = rvs[b * kk]

                def radd(i, cr):
                    for col in range(d // 16):
                        sl = pl.ds(col * 16, 16)
                        s = acc[i, sl]
                        for j in range(1, kk):
                            s = s + rvs[b * kk + j][i, sl]
                        acc[i, sl] = s
                    return cr

                lax.fori_loop(0, chunk, radd, 0)
                pltpu.sync_copy(acc, out_hbm.at[pl.ds(base + c * chunk, chunk)])
            return carry

        lax.fori_loop(0, nch // 2, body, 0)

    return k(table, idxs)


def kernel(x, expert_indices, expert_weights, W_up, W_down):
    tsteps, n, d = x.shape
    e, f, _ = W_up.shape
    k = expert_indices.shape[1]
    mb = _MB

    row_src, row_w, q, block_expert, block_valid, r_pad, g = _routing(
        expert_indices, expert_weights, e, mb)

    x2 = x.reshape(tsteps * n, d)
    xg_idx = ((jnp.arange(tsteps, dtype=jnp.int32) * n)[:, None]
              + row_src[None, :]).reshape(-1)
    xg = _sc_gather(x2, xg_idx, 32).reshape(tsteps, r_pad, d)

    w_exp = jnp.broadcast_to(row_w[:, None], (r_pad, 128))

    grid_spec = pltpu.PrefetchScalarGridSpec(
        num_scalar_prefetch=2,
        grid=(g,),
        in_specs=[
            pl.BlockSpec((tsteps, mb, d), lambda i, be, bv: (0, i, 0)),
            pl.BlockSpec((1, f, d), lambda i, be, bv: (be[i], 0, 0)),
            pl.BlockSpec((1, d, f), lambda i, be, bv: (be[i], 0, 0)),
            pl.BlockSpec((mb, 128), lambda i, be, bv: (i, 0)),
        ],
        out_specs=pl.BlockSpec((tsteps, mb, d), lambda i, be, bv: (0, i, 0)),
    )
    og = pl.pallas_call(
        _ffn_body,
        grid_spec=grid_spec,
        out_shape=jax.ShapeDtypeStruct((tsteps, r_pad, d), jnp.float32),
    )(block_expert, block_valid, xg, W_up, W_down, w_exp)

    og2 = og.reshape(tsteps * r_pad, d)
    c_idx = jnp.stack([
        ((jnp.arange(tsteps, dtype=jnp.int32) * r_pad)[:, None]
         + q[None, :, j]).reshape(-1)
        for j in range(k)])
    out2 = _sc_combine(og2, c_idx, 16)
    return out2.reshape(tsteps, n, d)


# speedup vs baseline: 5.5375x; 1.4399x over previous
import functools

import jax
import jax.numpy as jnp
from jax import lax
from jax.experimental import pallas as pl
from jax.experimental.pallas import tpu as pltpu
from jax.experimental.pallas import tpu_sc as plsc

_BETA = 0.9
_VTH = 1.0
_MB = 128


def _ffn_body(be_ref, bv_ref, xg_ref, wup_ref, wdn_ref, w_ref, out_ref):
    i = pl.program_id(0)

    @pl.when(bv_ref[i] == 1)
    def _():
        wup = wup_ref[0]
        wdn = wdn_ref[0]
        wrow = w_ref[:, 0:1]
        tsteps, mb, _ = xg_ref.shape
        f = wup.shape[0]
        dm = wdn.shape[0]
        v1 = jnp.zeros((mb, f), jnp.float32)
        v2 = jnp.zeros((mb, dm), jnp.float32)
        for t in range(tsteps):
            h = jax.lax.dot_general(
                xg_ref[t], wup, (((1,), (1,)), ((), ())),
                preferred_element_type=jnp.float32)
            v1 = _BETA * v1 + h
            s1 = (v1 >= _VTH).astype(jnp.float32)
            v1 = v1 - s1 * _VTH
            o = jax.lax.dot_general(
                s1, wdn, (((1,), (1,)), ((), ())),
                preferred_element_type=jnp.float32)
            v2 = _BETA * v2 + o
            s2 = (v2 >= _VTH).astype(jnp.float32)
            v2 = v2 - s2 * _VTH
            out_ref[t] = s2 * wrow


def _routing(expert_indices, expert_weights, n_experts, mb):
    n, k = expert_indices.shape
    p = n * k
    flat_e = expert_indices.reshape(-1).astype(jnp.int32)
    w_flat = expert_weights.reshape(-1)
    onehot = (flat_e[:, None] == jnp.arange(n_experts, dtype=jnp.int32)[None, :]
              ).astype(jnp.int32)
    csum = jnp.cumsum(onehot, axis=0)
    counts = csum[-1]
    rank = jnp.take_along_axis(csum, flat_e[:, None], axis=1)[:, 0] - 1
    padded = ((counts + mb - 1) // mb) * mb
    ends = jnp.cumsum(padded)
    starts = ends - padded
    r_pad = p + n_experts * mb
    g = r_pad // mb
    q_flat = starts[flat_e] + rank
    row_w = jnp.zeros((r_pad,), jnp.float32).at[q_flat].set(w_flat)
    q = q_flat.reshape(n, k)
    gstart = jnp.arange(g, dtype=jnp.int32) * mb
    block_expert = jnp.minimum(
        jnp.sum((gstart[:, None] >= ends[None, :]).astype(jnp.int32), axis=1),
        n_experts - 1).astype(jnp.int32)
    block_valid = (gstart < ends[-1]).astype(jnp.int32)
    return row_w, q, block_expert, block_valid, r_pad, g


def _sc_scatter_dispatch(x2, idxs, nrows_out, chunk):
    kk, nsrc = idxs.shape
    d = x2.shape[1]
    info = plsc.get_sparse_core_info()
    nc = info.num_cores
    nw = nc * info.num_subcores
    per_w = nsrc // nw
    assert per_w * nw == nsrc and per_w % (2 * chunk) == 0
    nch = per_w // chunk
    mesh = plsc.VectorSubcoreMesh(core_axis_name="c", subcore_axis_name="s")

    @functools.partial(
        pl.kernel, mesh=mesh,
        out_type=jax.ShapeDtypeStruct((nrows_out, d), jnp.float32),
        scratch_types=(
            [pltpu.VMEM((chunk, d), jnp.float32) for _ in range(2)]
            + [pltpu.VMEM((chunk,), jnp.int32) for _ in range(2 * kk)]
            + [pltpu.SemaphoreType.DMA for _ in range(2)]
            + [pltpu.SemaphoreType.DMA for _ in range(2 * kk)]
        ))
    def k(x_hbm, idx_hbm, out_hbm, *scratch):
        rows = scratch[:2]
        ivs = scratch[2:2 + 2 * kk]
        ldsem = scratch[2 + 2 * kk:4 + 2 * kk]
        scsem = scratch[4 + 2 * kk:]
        wid = lax.axis_index("s") * nc + lax.axis_index("c")
        base = wid * per_w

        def load(c, b):
            pltpu.async_copy(x_hbm.at[pl.ds(base + c * chunk, chunk)],
                             rows[b], ldsem[b])

        load(0, 0)

        def body(go, carry):
            for b in range(2):
                c = go * 2 + b
                pltpu.make_async_copy(
                    x_hbm.at[pl.ds(base + c * chunk, chunk)],
                    rows[b], ldsem[b]).wait()
                for j in range(kk):
                    iv = ivs[b * kk + j]
                    pltpu.sync_copy(
                        idx_hbm.at[j, pl.ds(base + c * chunk, chunk)], iv)
                    pltpu.async_copy(rows[b], out_hbm.at[iv],
                                     scsem[b * kk + j])

                @pl.when(c + 1 < nch)
                def _():
                    ob = 1 - b
                    for j in range(kk):
                        @pl.when(c >= 1)
                        def _():
                            pltpu.make_async_copy(
                                rows[ob], out_hbm.at[ivs[ob * kk + j]],
                                scsem[ob * kk + j]).wait()
                    load(c + 1, ob)
            return carry

        lax.fori_loop(0, nch // 2, body, 0)
        lb = (nch - 1) % 2
        for j in range(kk):
            pltpu.make_async_copy(rows[lb], out_hbm.at[ivs[lb * kk + j]],
                                  scsem[lb * kk + j]).wait()

    return k(x2, idxs)


def _sc_combine(table, idxs, chunk):
    kk, nrows = idxs.shape
    d = table.shape[1]
    info = plsc.get_sparse_core_info()
    nc = info.num_cores
    nw = nc * info.num_subcores
    per_w = nrows // nw
    assert per_w * nw == nrows and per_w % (2 * chunk) == 0
    nch = per_w // chunk
    mesh = plsc.VectorSubcoreMesh(core_axis_name="c", subcore_axis_name="s")

    @functools.partial(
        pl.kernel, mesh=mesh,
        out_type=jax.ShapeDtypeStruct((nrows, d), jnp.float32),
        scratch_types=(
            [pltpu.VMEM((chunk,), jnp.int32) for _ in range(2 * kk)]
            + [pltpu.VMEM((chunk, d), jnp.float32) for _ in range(2 * kk)]
            + [pltpu.SemaphoreType.DMA for _ in range(2 * kk)]
        ))
    def k(table_hbm, idx_hbm, out_hbm, *scratch):
        ivs = scratch[:2 * kk]
        rvs = scratch[2 * kk:4 * kk]
        sems = scratch[4 * kk:]
        wid = lax.axis_index("s") * nc + lax.axis_index("c")
        base = wid * per_w

        def issue(c, b):
            for j in range(kk):
                iv = ivs[b * kk + j]
                rv = rvs[b * kk + j]
                sem = sems[b * kk + j]
                pltpu.sync_copy(idx_hbm.at[j, pl.ds(base + c * chunk, chunk)], iv)
                pltpu.async_copy(table_hbm.at[iv], rv, sem)

        issue(0, 0)

        def body(go, carry):
            for b in range(2):
                c = go * 2 + b
                for j in range(kk):
                    pltpu.make_async_copy(
                        table_hbm.at[ivs[b * kk + j]], rvs[b * kk + j],
                        sems[b * kk + j]).wait()

                @pl.when(c + 1 < nch)
                def _():
                    issue(c + 1, 1 - b)

                acc = rvs[b * kk]

                def radd(i, cr):
                    for col in range(d // 16):
                        sl = pl.ds(col * 16, 16)
                        s = acc[i, sl]
                        for j in range(1, kk):
                            s = s + rvs[b * kk + j][i, sl]
                        acc[i, sl] = s
                    return cr

                lax.fori_loop(0, chunk, radd, 0)
                pltpu.sync_copy(acc, out_hbm.at[pl.ds(base + c * chunk, chunk)])
            return carry

        lax.fori_loop(0, nch // 2, body, 0)

    return k(table, idxs)


def kernel(x, expert_indices, expert_weights, W_up, W_down):
    tsteps, n, d = x.shape
    e, f, _ = W_up.shape
    k = expert_indices.shape[1]
    mb = _MB

    row_w, q, block_expert, block_valid, r_pad, g = _routing(
        expert_indices, expert_weights, e, mb)

    c_idx = jnp.stack([
        ((jnp.arange(tsteps, dtype=jnp.int32) * r_pad)[:, None]
         + q[None, :, j]).reshape(-1)
        for j in range(k)])

    x2 = x.reshape(tsteps * n, d)
    xg = _sc_scatter_dispatch(x2, c_idx, tsteps * r_pad, 32
                              ).reshape(tsteps, r_pad, d)

    w_exp = jnp.broadcast_to(row_w[:, None], (r_pad, 128))

    grid_spec = pltpu.PrefetchScalarGridSpec(
        num_scalar_prefetch=2,
        grid=(g,),
        in_specs=[
            pl.BlockSpec((tsteps, mb, d), lambda i, be, bv: (0, i, 0)),
            pl.BlockSpec((1, f, d), lambda i, be, bv: (be[i], 0, 0)),
            pl.BlockSpec((1, d, f), lambda i, be, bv: (be[i], 0, 0)),
            pl.BlockSpec((mb, 128), lambda i, be, bv: (i, 0)),
        ],
        out_specs=pl.BlockSpec((tsteps, mb, d), lambda i, be, bv: (0, i, 0)),
    )
    og = pl.pallas_call(
        _ffn_body,
        grid_spec=grid_spec,
        out_shape=jax.ShapeDtypeStruct((tsteps, r_pad, d), jnp.float32),
    )(block_expert, block_valid, xg, W_up, W_down, w_exp)

    og2 = og.reshape(tsteps * r_pad, d)
    out2 = _sc_combine(og2, c_idx, 16)
    return out2.reshape(tsteps, n, d)


# speedup vs baseline: 6.5765x; 1.1876x over previous
import functools

import jax
import jax.numpy as jnp
from jax import lax
from jax.experimental import pallas as pl
from jax.experimental.pallas import tpu as pltpu
from jax.experimental.pallas import tpu_sc as plsc

_BETA = 0.9
_VTH = 1.0
_MB = 128


def _ffn_body(be_ref, bv_ref, xg_ref, wup_ref, wdn_ref, w_ref, out_ref):
    i = pl.program_id(0)

    @pl.when(bv_ref[i] == 1)
    def _():
        wup = wup_ref[0]
        wdn = wdn_ref[0]
        wrow = w_ref[:, 0:1]
        tsteps, mb, dm = xg_ref.shape
        f = wup.shape[0]
        h_all = jax.lax.dot_general(
            xg_ref[...].reshape(tsteps * mb, dm), wup,
            (((1,), (1,)), ((), ())),
            preferred_element_type=jnp.float32)
        v1 = jnp.zeros((mb, f), jnp.float32)
        spikes = []
        for t in range(tsteps):
            v1 = _BETA * v1 + h_all[t * mb:(t + 1) * mb]
            s1 = (v1 >= _VTH).astype(jnp.float32)
            v1 = v1 - s1 * _VTH
            spikes.append(s1)
        s_all = jnp.concatenate(spikes, axis=0)
        o_all = jax.lax.dot_general(
            s_all, wdn, (((1,), (1,)), ((), ())),
            preferred_element_type=jnp.float32)
        v2 = jnp.zeros((mb, dm), jnp.float32)
        for t in range(tsteps):
            v2 = _BETA * v2 + o_all[t * mb:(t + 1) * mb]
            s2 = (v2 >= _VTH).astype(jnp.float32)
            v2 = v2 - s2 * _VTH
            out_ref[t] = s2 * wrow


def _routing(expert_indices, expert_weights, n_experts, mb):
    n, k = expert_indices.shape
    p = n * k
    flat_e = expert_indices.reshape(-1).astype(jnp.int32)
    w_flat = expert_weights.reshape(-1)
    onehot = (flat_e[:, None] == jnp.arange(n_experts, dtype=jnp.int32)[None, :]
              ).astype(jnp.int32)
    csum = jnp.cumsum(onehot, axis=0)
    counts = csum[-1]
    rank = jnp.take_along_axis(csum, flat_e[:, None], axis=1)[:, 0] - 1
    padded = ((counts + mb - 1) // mb) * mb
    ends = jnp.cumsum(padded)
    starts = ends - padded
    r_pad = p + n_experts * mb
    g = r_pad // mb
    q_flat = starts[flat_e] + rank
    row_w = jnp.zeros((r_pad,), jnp.float32).at[q_flat].set(w_flat)
    q = q_flat.reshape(n, k)
    gstart = jnp.arange(g, dtype=jnp.int32) * mb
    block_expert = jnp.minimum(
        jnp.sum((gstart[:, None] >= ends[None, :]).astype(jnp.int32), axis=1),
        n_experts - 1).astype(jnp.int32)
    block_valid = (gstart < ends[-1]).astype(jnp.int32)
    return row_w, q, block_expert, block_valid, r_pad, g


def _sc_scatter_dispatch(x2, idxs, nrows_out, chunk):
    kk, nsrc = idxs.shape
    d = x2.shape[1]
    info = plsc.get_sparse_core_info()
    nc = info.num_cores
    nw = nc * info.num_subcores
    per_w = nsrc // nw
    assert per_w * nw == nsrc and per_w % (2 * chunk) == 0
    nch = per_w // chunk
    mesh = plsc.VectorSubcoreMesh(core_axis_name="c", subcore_axis_name="s")

    @functools.partial(
        pl.kernel, mesh=mesh,
        out_type=jax.ShapeDtypeStruct((nrows_out, d), jnp.float32),
        scratch_types=(
            [pltpu.VMEM((chunk, d), jnp.float32) for _ in range(2)]
            + [pltpu.VMEM((chunk,), jnp.int32) for _ in range(2 * kk)]
            + [pltpu.SemaphoreType.DMA for _ in range(2)]
            + [pltpu.SemaphoreType.DMA for _ in range(2 * kk)]
        ))
    def k(x_hbm, idx_hbm, out_hbm, *scratch):
        rows = scratch[:2]
        ivs = scratch[2:2 + 2 * kk]
        ldsem = scratch[2 + 2 * kk:4 + 2 * kk]
        scsem = scratch[4 + 2 * kk:]
        wid = lax.axis_index("s") * nc + lax.axis_index("c")
        base = wid * per_w

        def load(c, b):
            pltpu.async_copy(x_hbm.at[pl.ds(base + c * chunk, chunk)],
                             rows[b], ldsem[b])

        load(0, 0)

        def body(go, carry):
            for b in range(2):
                c = go * 2 + b
                pltpu.make_async_copy(
                    x_hbm.at[pl.ds(base + c * chunk, chunk)],
                    rows[b], ldsem[b]).wait()
                for j in range(kk):
                    iv = ivs[b * kk + j]
                    pltpu.sync_copy(
                        idx_hbm.at[j, pl.ds(base + c * chunk, chunk)], iv)
                    pltpu.async_copy(rows[b], out_hbm.at[iv],
                                     scsem[b * kk + j])

                @pl.when(c + 1 < nch)
                def _():
                    ob = 1 - b
                    for j in range(kk):
                        @pl.when(c >= 1)
                        def _():
                            pltpu.make_async_copy(
                                rows[ob], out_hbm.at[ivs[ob * kk + j]],
                                scsem[ob * kk + j]).wait()
                    load(c + 1, ob)
            return carry

        lax.fori_loop(0, nch // 2, body, 0)
        lb = (nch - 1) % 2
        for j in range(kk):
            pltpu.make_async_copy(rows[lb], out_hbm.at[ivs[lb * kk + j]],
                                  scsem[lb * kk + j]).wait()

    return k(x2, idxs)


def _sc_combine(table, idxs, chunk):
    kk, nrows = idxs.shape
    d = table.shape[1]
    info = plsc.get_sparse_core_info()
    nc = info.num_cores
    nw = nc * info.num_subcores
    per_w = nrows // nw
    assert per_w * nw == nrows and per_w % (2 * chunk) == 0
    nch = per_w // chunk
    mesh = plsc.VectorSubcoreMesh(core_axis_name="c", subcore_axis_name="s")

    @functools.partial(
        pl.kernel, mesh=mesh,
        out_type=jax.ShapeDtypeStruct((nrows, d), jnp.float32),
        scratch_types=(
            [pltpu.VMEM((chunk,), jnp.int32) for _ in range(2 * kk)]
            + [pltpu.VMEM((chunk, d), jnp.float32) for _ in range(2 * kk)]
            + [pltpu.SemaphoreType.DMA for _ in range(2 * kk)]
        ))
    def k(table_hbm, idx_hbm, out_hbm, *scratch):
        ivs = scratch[:2 * kk]
        rvs = scratch[2 * kk:4 * kk]
        sems = scratch[4 * kk:]
        wid = lax.axis_index("s") * nc + lax.axis_index("c")
        base = wid * per_w

        def issue(c, b):
            for j in range(kk):
                iv = ivs[b * kk + j]
                rv = rvs[b * kk + j]
                sem = sems[b * kk + j]
                pltpu.sync_copy(idx_hbm.at[j, pl.ds(base + c * chunk, chunk)], iv)
                pltpu.async_copy(table_hbm.at[iv], rv, sem)

        issue(0, 0)

        def body(go, carry):
            for b in range(2):
                c = go * 2 + b
                for j in range(kk):
                    pltpu.make_async_copy(
                        table_hbm.at[ivs[b * kk + j]], rvs[b * kk + j],
                        sems[b * kk + j]).wait()

                @pl.when(c + 1 < nch)
                def _():
                    issue(c + 1, 1 - b)

                acc = rvs[b * kk]

                def radd(i, cr):
                    for col in range(d // 16):
                        sl = pl.ds(col * 16, 16)
                        s = acc[i, sl]
                        for j in range(1, kk):
                            s = s + rvs[b * kk + j][i, sl]
                        acc[i, sl] = s
                    return cr

                lax.fori_loop(0, chunk, radd, 0)
                pltpu.sync_copy(acc, out_hbm.at[pl.ds(base + c * chunk, chunk)])
            return carry

        lax.fori_loop(0, nch // 2, body, 0)

    return k(table, idxs)


def kernel(x, expert_indices, expert_weights, W_up, W_down):
    tsteps, n, d = x.shape
    e, f, _ = W_up.shape
    k = expert_indices.shape[1]
    mb = _MB

    row_w, q, block_expert, block_valid, r_pad, g = _routing(
        expert_indices, expert_weights, e, mb)

    c_idx = jnp.stack([
        ((jnp.arange(tsteps, dtype=jnp.int32) * r_pad)[:, None]
         + q[None, :, j]).reshape(-1)
        for j in range(k)])

    x2 = x.reshape(tsteps * n, d)
    xg = _sc_scatter_dispatch(x2, c_idx, tsteps * r_pad, 32
                              ).reshape(tsteps, r_pad, d)

    w_exp = jnp.broadcast_to(row_w[:, None], (r_pad, 128))

    grid_spec = pltpu.PrefetchScalarGridSpec(
        num_scalar_prefetch=2,
        grid=(g,),
        in_specs=[
            pl.BlockSpec((tsteps, mb, d), lambda i, be, bv: (0, i, 0)),
            pl.BlockSpec((1, f, d), lambda i, be, bv: (be[i], 0, 0)),
            pl.BlockSpec((1, d, f), lambda i, be, bv: (be[i], 0, 0)),
            pl.BlockSpec((mb, 128), lambda i, be, bv: (i, 0)),
        ],
        out_specs=pl.BlockSpec((tsteps, mb, d), lambda i, be, bv: (0, i, 0)),
    )
    og = pl.pallas_call(
        _ffn_body,
        grid_spec=grid_spec,
        out_shape=jax.ShapeDtypeStruct((tsteps, r_pad, d), jnp.float32),
    )(block_expert, block_valid, xg, W_up, W_down, w_exp)

    og2 = og.reshape(tsteps * r_pad, d)
    out2 = _sc_combine(og2, c_idx, 16)
    return out2.reshape(tsteps, n, d)


# speedup vs baseline: 6.6504x; 1.0112x over previous
import functools

import jax
import jax.numpy as jnp
from jax import lax
from jax.experimental import pallas as pl
from jax.experimental.pallas import tpu as pltpu
from jax.experimental.pallas import tpu_sc as plsc

_BETA = 0.9
_VTH = 1.0
_MB = 128


def _ffn_body(be_ref, bv_ref, xg_ref, wup_ref, wdn_ref, w_ref, out_ref):
    i = pl.program_id(0)

    @pl.when(bv_ref[i] == 1)
    def _():
        wup = wup_ref[0]
        wdn = wdn_ref[0]
        wrow = w_ref[:, 0:1]
        tsteps, mb, dm = xg_ref.shape
        f = wup.shape[0]
        h_all = jax.lax.dot_general(
            xg_ref[...].reshape(tsteps * mb, dm), wup,
            (((1,), (1,)), ((), ())),
            preferred_element_type=jnp.float32)
        v1 = jnp.zeros((mb, f), jnp.float32)
        spikes = []
        for t in range(tsteps):
            v1 = _BETA * v1 + h_all[t * mb:(t + 1) * mb]
            s1 = (v1 >= _VTH).astype(jnp.float32)
            v1 = v1 - s1 * _VTH
            spikes.append(s1)
        s_all = jnp.concatenate(spikes, axis=0)
        o_all = jax.lax.dot_general(
            s_all, wdn, (((1,), (1,)), ((), ())),
            preferred_element_type=jnp.float32)
        v2 = jnp.zeros((mb, dm), jnp.float32)
        for t in range(tsteps):
            v2 = _BETA * v2 + o_all[t * mb:(t + 1) * mb]
            s2 = (v2 >= _VTH).astype(jnp.float32)
            v2 = v2 - s2 * _VTH
            out_ref[t] = s2 * wrow


def _routing(expert_indices, expert_weights, n_experts, mb):
    n, k = expert_indices.shape
    p = n * k
    flat_e = expert_indices.reshape(-1).astype(jnp.int32)
    w_flat = expert_weights.reshape(-1)
    onehot = (flat_e[:, None] == jnp.arange(n_experts, dtype=jnp.int32)[None, :]
              ).astype(jnp.int32)
    csum = jnp.cumsum(onehot, axis=0)
    counts = csum[-1]
    rank = jnp.take_along_axis(csum, flat_e[:, None], axis=1)[:, 0] - 1
    padded = ((counts + mb - 1) // mb) * mb
    ends = jnp.cumsum(padded)
    starts = ends - padded
    r_pad = p + n_experts * mb
    g = r_pad // mb
    q_flat = starts[flat_e] + rank
    row_w = jnp.zeros((r_pad,), jnp.float32).at[q_flat].set(w_flat)
    q = q_flat.reshape(n, k)
    gstart = jnp.arange(g, dtype=jnp.int32) * mb
    block_expert = jnp.minimum(
        jnp.sum((gstart[:, None] >= ends[None, :]).astype(jnp.int32), axis=1),
        n_experts - 1).astype(jnp.int32)
    block_valid = (gstart < ends[-1]).astype(jnp.int32)
    return row_w, q, block_expert, block_valid, r_pad, g


def _sc_scatter_dispatch(x2, idxs, nrows_out, chunk):
    kk, nsrc = idxs.shape
    d = x2.shape[1]
    info = plsc.get_sparse_core_info()
    nc = info.num_cores
    nw = nc * info.num_subcores
    per_w = nsrc // nw
    assert per_w * nw == nsrc and per_w % (2 * chunk) == 0
    nch = per_w // chunk
    mesh = plsc.VectorSubcoreMesh(core_axis_name="c", subcore_axis_name="s")

    @functools.partial(
        pl.kernel, mesh=mesh,
        out_type=jax.ShapeDtypeStruct((nrows_out, d), jnp.float32),
        scratch_types=(
            [pltpu.VMEM((chunk, d), jnp.float32) for _ in range(2)]
            + [pltpu.VMEM((chunk,), jnp.int32) for _ in range(2 * kk)]
            + [pltpu.SemaphoreType.DMA for _ in range(2)]
            + [pltpu.SemaphoreType.DMA for _ in range(2 * kk)]
        ))
    def k(x_hbm, idx_hbm, out_hbm, *scratch):
        rows = scratch[:2]
        ivs = scratch[2:2 + 2 * kk]
        ldsem = scratch[2 + 2 * kk:4 + 2 * kk]
        scsem = scratch[4 + 2 * kk:]
        wid = lax.axis_index("s") * nc + lax.axis_index("c")
        base = wid * per_w

        def load(c, b):
            pltpu.async_copy(x_hbm.at[pl.ds(base + c * chunk, chunk)],
                             rows[b], ldsem[b])

        load(0, 0)

        def body(go, carry):
            for b in range(2):
                c = go * 2 + b
                pltpu.make_async_copy(
                    x_hbm.at[pl.ds(base + c * chunk, chunk)],
                    rows[b], ldsem[b]).wait()
                for j in range(kk):
                    iv = ivs[b * kk + j]
                    pltpu.sync_copy(
                        idx_hbm.at[j, pl.ds(base + c * chunk, chunk)], iv)
                    pltpu.async_copy(rows[b], out_hbm.at[iv],
                                     scsem[b * kk + j])

                @pl.when(c + 1 < nch)
                def _():
                    ob = 1 - b
                    for j in range(kk):
                        @pl.when(c >= 1)
                        def _():
                            pltpu.make_async_copy(
                                rows[ob], out_hbm.at[ivs[ob * kk + j]],
                                scsem[ob * kk + j]).wait()
                    load(c + 1, ob)
            return carry

        lax.fori_loop(0, nch // 2, body, 0)
        lb = (nch - 1) % 2
        for j in range(kk):
            pltpu.make_async_copy(rows[lb], out_hbm.at[ivs[lb * kk + j]],
                                  scsem[lb * kk + j]).wait()

    return k(x2, idxs)


def _sc_combine(table, idxs, chunk):
    kk, nrows = idxs.shape
    d = table.shape[1]
    info = plsc.get_sparse_core_info()
    nc = info.num_cores
    nw = nc * info.num_subcores
    per_w = nrows // nw
    assert per_w * nw == nrows and per_w % (2 * chunk) == 0
    nch = per_w // chunk
    mesh = plsc.VectorSubcoreMesh(core_axis_name="c", subcore_axis_name="s")

    @functools.partial(
        pl.kernel, mesh=mesh,
        out_type=jax.ShapeDtypeStruct((nrows, d), jnp.float32),
        scratch_types=(
            [pltpu.VMEM((chunk,), jnp.int32) for _ in range(2 * kk)]
            + [pltpu.VMEM((chunk, d), jnp.float32) for _ in range(2 * kk)]
            + [pltpu.SemaphoreType.DMA for _ in range(2 * kk)]
        ))
    def k(table_hbm, idx_hbm, out_hbm, *scratch):
        ivs = scratch[:2 * kk]
        rvs = scratch[2 * kk:4 * kk]
        sems = scratch[4 * kk:]
        wid = lax.axis_index("s") * nc + lax.axis_index("c")
        base = wid * per_w

        def issue(c, b):
            for j in range(kk):
                iv = ivs[b * kk + j]
                rv = rvs[b * kk + j]
                sem = sems[b * kk + j]
                pltpu.sync_copy(idx_hbm.at[j, pl.ds(base + c * chunk, chunk)], iv)
                pltpu.async_copy(table_hbm.at[iv], rv, sem)

        issue(0, 0)

        def body(go, carry):
            for b in range(2):
                c = go * 2 + b
                for j in range(kk):
                    pltpu.make_async_copy(
                        table_hbm.at[ivs[b * kk + j]], rvs[b * kk + j],
                        sems[b * kk + j]).wait()

                @pl.when(c + 1 < nch)
                def _():
                    issue(c + 1, 1 - b)

                acc = rvs[b * kk]

                def radd(i, cr):
                    for col in range(d // 16):
                        sl = pl.ds(col * 16, 16)
                        s = acc[i, sl]
                        for j in range(1, kk):
                            s = s + rvs[b * kk + j][i, sl]
                        acc[i, sl] = s
                    return cr

                lax.fori_loop(0, chunk, radd, 0)
                pltpu.sync_copy(acc, out_hbm.at[pl.ds(base + c * chunk, chunk)])
            return carry

        lax.fori_loop(0, nch // 2, body, 0)

    return k(table, idxs)


def kernel(x, expert_indices, expert_weights, W_up, W_down):
    tsteps, n, d = x.shape
    e, f, _ = W_up.shape
    k = expert_indices.shape[1]
    mb = _MB

    row_w, q, block_expert, block_valid, r_pad, g = _routing(
        expert_indices, expert_weights, e, mb)

    c_idx = jnp.stack([
        ((jnp.arange(tsteps, dtype=jnp.int32) * r_pad)[:, None]
         + q[None, :, j]).reshape(-1)
        for j in range(k)])

    x2 = x.reshape(tsteps * n, d)
    xg = _sc_scatter_dispatch(x2, c_idx, tsteps * r_pad, 32
                              ).reshape(tsteps, r_pad, d)

    w_exp = jnp.broadcast_to(row_w[:, None], (r_pad, 128))

    grid_spec = pltpu.PrefetchScalarGridSpec(
        num_scalar_prefetch=2,
        grid=(g,),
        in_specs=[
            pl.BlockSpec((tsteps, mb, d),
                         lambda i, be, bv: (0, jnp.where(bv[i] == 1, i, 0), 0)),
            pl.BlockSpec((1, f, d), lambda i, be, bv: (be[i], 0, 0)),
            pl.BlockSpec((1, d, f), lambda i, be, bv: (be[i], 0, 0)),
            pl.BlockSpec((mb, 128),
                         lambda i, be, bv: (jnp.where(bv[i] == 1, i, 0), 0)),
        ],
        out_specs=pl.BlockSpec(
            (tsteps, mb, d),
            lambda i, be, bv: (0, jnp.where(bv[i] == 1, i, g - 1), 0)),
    )
    og = pl.pallas_call(
        _ffn_body,
        grid_spec=grid_spec,
        out_shape=jax.ShapeDtypeStruct((tsteps, r_pad, d), jnp.float32),
    )(block_expert, block_valid, xg, W_up, W_down, w_exp)

    og2 = og.reshape(tsteps * r_pad, d)
    out2 = _sc_combine(og2, c_idx, 16)
    return out2.reshape(tsteps, n, d)
